# baseline (speedup 1.0000x reference)
import functools

import jax
import jax.numpy as jnp
from jax import lax
from jax.experimental import pallas as pl
from jax.experimental.pallas import tpu as pltpu
from jax.experimental.pallas import tpu_sc as plsc

B = 16384
V = 1000000
E = 64
CTX = 5
H = 256
OUT = 50

N = B * CTX


@functools.lru_cache(maxsize=1)
def _make_sc_gather():
    info = plsc.get_sparse_core_info()
    NC, NS = info.num_cores, info.num_subcores
    NW = NC * NS
    n_per_w = N // NW
    CH = 512
    n_ch = n_per_w // CH

    mesh = plsc.VectorSubcoreMesh(core_axis_name="c", subcore_axis_name="s")

    @functools.partial(
        pl.kernel,
        mesh=mesh,
        compiler_params=pltpu.CompilerParams(use_tc_tiling_on_sc=False),
        out_type=jax.ShapeDtypeStruct((N, E), jnp.float32),
        scratch_types=[
            pltpu.VMEM((n_per_w,), jnp.int32),
            pltpu.VMEM((CH, E), jnp.float32),
            pltpu.VMEM((CH, E), jnp.float32),
            pltpu.SemaphoreType.DMA,
            pltpu.SemaphoreType.DMA,
        ],
    )
    def gather_k(table_hbm, idx_hbm, out_hbm, idx_v, buf0, buf1, sem0, sem1):
        wid = lax.axis_index("s") * NC + lax.axis_index("c")
        base = wid * n_per_w
        pltpu.sync_copy(idx_hbm.at[pl.ds(base, n_per_w)], idx_v)
        bufs = (buf0, buf1)
        sems = (sem0, sem1)
        copies = [None, None]
        for ci in range(n_ch):
            s = ci % 2
            copies[s] = pltpu.async_copy(
                table_hbm.at[idx_v.at[pl.ds(ci * CH, CH)]], bufs[s], sems[s]
            )
            if ci > 0:
                p = (ci - 1) % 2
                copies[p].wait()
                pltpu.sync_copy(bufs[p], out_hbm.at[pl.ds(base + (ci - 1) * CH, CH)])
        last = (n_ch - 1) % 2
        copies[last].wait()
        pltpu.sync_copy(bufs[last], out_hbm.at[pl.ds(base + (n_ch - 1) * CH, CH)])

    return gather_k


_BLK = 2048


def _mlp_body(flat_ref, x_ref, t0_ref, w1_ref, b1_ref, w2_ref, b2_ref, out_ref):
    corr = jnp.concatenate(
        [
            -jnp.dot(
                t0_ref[...],
                w1_ref[c * E : (c + 1) * E, :],
                preferred_element_type=jnp.float32,
            )
            for c in range(CTX)
        ],
        axis=0,
    )
    padm = (x_ref[...] == 0).astype(jnp.float32)
    acc = jnp.dot(flat_ref[...], w1_ref[...], preferred_element_type=jnp.float32)
    acc = acc + jnp.dot(padm, corr, preferred_element_type=jnp.float32)
    h = jnp.tanh(acc + b1_ref[...])
    out_ref[...] = (
        jnp.dot(h, w2_ref[...], preferred_element_type=jnp.float32) + b2_ref[...]
    )


def _mlp(flat, x32, t0, W1, b1, W2, b2):
    grid = (B // _BLK,)
    return pl.pallas_call(
        _mlp_body,
        grid=grid,
        in_specs=[
            pl.BlockSpec((_BLK, E * CTX), lambda i: (i, 0)),
            pl.BlockSpec((_BLK, CTX), lambda i: (i, 0)),
            pl.BlockSpec((1, E), lambda i: (0, 0)),
            pl.BlockSpec((E * CTX, H), lambda i: (0, 0)),
            pl.BlockSpec((1, H), lambda i: (0, 0)),
            pl.BlockSpec((H, OUT), lambda i: (0, 0)),
            pl.BlockSpec((1, OUT), lambda i: (0, 0)),
        ],
        out_specs=pl.BlockSpec((_BLK, OUT), lambda i: (i, 0)),
        out_shape=jax.ShapeDtypeStruct((B, OUT), jnp.float32),
    )(flat, x32, t0, W1, b1, W2, b2)


def kernel(x, table, W1, b1, W2, b2):
    x32 = x.astype(jnp.int32)
    idx = x32.reshape(-1)
    rows = _make_sc_gather()(table, idx)
    flat = rows.reshape(B, E * CTX)
    out = _mlp(
        flat, x32, table[0:1], W1, b1.reshape(1, H), W2, b2.reshape(1, OUT)
    )
    return out
